# baseline (speedup 1.0000x reference)
import functools

import jax
import jax.numpy as jnp
from jax import lax
from jax.experimental import pallas as pl
from jax.experimental.pallas import tpu as pltpu
from jax.experimental.pallas import tpu_sc as plsc

_N = 10000
_E = 320000
_D = 128
_DP = 160
_NC = 2
_NS = 16
_NW = _NC * _NS
_EPW = _E // _NW
_CH = 40
_NCH = _EPW // _CH
_NP = 10112
_RPT = _NP // _NS
_BM = 2000
_NBLK = _N // _BM


def _sc_agg_body(xa, src3, dst3, zsum, out_sum,
                 idx_s, idx_d, rows0, rows1, rows2, acc,
                 sem0, sem1, sem2, semz):
    c = lax.axis_index("c")
    s = lax.axis_index("s")
    wid = s * _NC + c

    def gather(j, buf, sem):
        pltpu.async_copy(xa.at[idx_s.at[j]], buf, sem)

    def wait_g(buf, sem):
        pltpu.make_async_copy(xa.at[idx_s.at[0]], buf, sem).wait()

    def scat(j, buf):
        pltpu.sync_copy(buf, acc.at[idx_d.at[j]], add=True)

    pltpu.sync_copy(src3.at[wid], idx_s)
    gather(0, rows0, sem0)
    gather(1, rows1, sem1)
    gather(2, rows2, sem2)
    pltpu.async_copy(dst3.at[wid], idx_d, semz)
    pltpu.async_copy(zsum, acc.at[pl.ds(s * _RPT, _RPT)], semz)
    pltpu.make_async_copy(dst3.at[wid], idx_d, semz).wait()
    pltpu.make_async_copy(zsum, acc.at[pl.ds(s * _RPT, _RPT)], semz).wait()
    plsc.subcore_barrier()

    def body(i, carry):
        j = 3 * i
        wait_g(rows0, sem0)
        scat(j, rows0)

        @pl.when(j + 3 < _NCH)
        def _pref0():
            gather(j + 3, rows0, sem0)

        wait_g(rows1, sem1)
        scat(j + 1, rows1)

        @pl.when(j + 4 < _NCH)
        def _pref1():
            gather(j + 4, rows1, sem1)

        wait_g(rows2, sem2)
        scat(j + 2, rows2)

        @pl.when(j + 5 < _NCH)
        def _pref2():
            gather(j + 5, rows2, sem2)

        return carry

    lax.fori_loop(0, _NCH // 3, body, 0)
    for r in range(_NCH - 3 * (_NCH // 3)):
        j = 3 * (_NCH // 3) + r
        buf, sem = [(rows0, sem0), (rows1, sem1), (rows2, sem2)][r]
        wait_g(buf, sem)
        scat(j, buf)

    plsc.subcore_barrier()
    pltpu.sync_copy(acc.at[pl.ds(s * _RPT, _RPT)],
                    out_sum.at[c, pl.ds(s * _RPT, _RPT)])


@functools.lru_cache(maxsize=1)
def _get_sc_agg():
    return pl.kernel(
        _sc_agg_body,
        out_type=jax.ShapeDtypeStruct((_NC, _NP, _DP), jnp.bfloat16),
        mesh=plsc.VectorSubcoreMesh(core_axis_name="c", subcore_axis_name="s"),
        scratch_types=[
            pltpu.VMEM((_NCH, _CH), jnp.int32),
            pltpu.VMEM((_NCH, _CH), jnp.int32),
            pltpu.VMEM((_CH, _DP), jnp.bfloat16),
            pltpu.VMEM((_CH, _DP), jnp.bfloat16),
            pltpu.VMEM((_CH, _DP), jnp.bfloat16),
            pltpu.VMEM_SHARED((_NP, _DP), jnp.bfloat16),
            pltpu.SemaphoreType.DMA,
            pltpu.SemaphoreType.DMA,
            pltpu.SemaphoreType.DMA,
            pltpu.SemaphoreType.DMA,
        ],
        compiler_params=pltpu.CompilerParams(use_tc_tiling_on_sc=False),
    )


def _tc_stats_body(nz_ref, x_ref, p_ref, w1_ref, w2_ref, b_ref, g_ref,
                   bt_ref, y_ref, st_ref, acc_ref):
    i = pl.program_id(0)

    @pl.when(i == 0)
    def _init():
        acc_ref[...] = jnp.zeros_like(acc_ref)
        st_ref[...] = jnp.zeros_like(st_ref)

    p0 = p_ref[0].astype(jnp.float32)
    p1 = p_ref[1].astype(jnp.float32)
    ssum = p0[:, :_D] + p1[:, :_D] + nz_ref[0, 0]
    cnt = jnp.maximum(p0[:, _D:_D + 1] + p1[:, _D:_D + 1], 1.0)
    mean = ssum / cnt
    y = (jnp.dot(x_ref[...], w1_ref[...], preferred_element_type=jnp.float32)
         + jnp.dot(mean, w2_ref[...], preferred_element_type=jnp.float32)
         + b_ref[...])
    y_ref[...] = y
    acc_ref[0:1, :] = acc_ref[0:1, :] + jnp.sum(y, axis=0, keepdims=True)
    acc_ref[1:2, :] = acc_ref[1:2, :] + jnp.sum(y * y, axis=0, keepdims=True)

    @pl.when(i == pl.num_programs(0) - 1)
    def _fin():
        mu = acc_ref[0:1, :] * (1.0 / _N)
        var = acc_ref[1:2, :] * (1.0 / _N) - mu * mu
        rstd = lax.rsqrt(var + 1e-5)
        scale = g_ref[...] * rstd
        st_ref[0:1, :] = scale
        st_ref[1:2, :] = bt_ref[...] - mu * scale


@functools.lru_cache(maxsize=1)
def _get_stats_call():
    return pl.pallas_call(
        _tc_stats_body,
        grid=(_NBLK,),
        in_specs=[
            pl.BlockSpec(memory_space=pltpu.SMEM),
            pl.BlockSpec((_BM, _D), lambda i: (i, 0)),
            pl.BlockSpec((_NC, _BM, _DP), lambda i: (0, i, 0)),
            pl.BlockSpec((_D, _D), lambda i: (0, 0)),
            pl.BlockSpec((_D, _D), lambda i: (0, 0)),
            pl.BlockSpec((1, _D), lambda i: (0, 0)),
            pl.BlockSpec((1, _D), lambda i: (0, 0)),
            pl.BlockSpec((1, _D), lambda i: (0, 0)),
        ],
        out_specs=[
            pl.BlockSpec((_BM, _D), lambda i: (i, 0)),
            pl.BlockSpec((8, _D), lambda i: (0, 0)),
        ],
        out_shape=[
            jax.ShapeDtypeStruct((_N, _D), jnp.float32),
            jax.ShapeDtypeStruct((8, _D), jnp.float32),
        ],
        scratch_shapes=[pltpu.VMEM((8, _D), jnp.float32)],
        compiler_params=pltpu.CompilerParams(
            dimension_semantics=("arbitrary",)),
    )


def _tc_norm_body(y_ref, st_ref, o_ref):
    o_ref[...] = jnp.maximum(y_ref[...] * st_ref[0:1, :] + st_ref[1:2, :], 0.0)


@functools.lru_cache(maxsize=1)
def _get_norm_call():
    return pl.pallas_call(
        _tc_norm_body,
        grid=(_NBLK,),
        in_specs=[
            pl.BlockSpec((_BM, _D), lambda i: (i, 0)),
            pl.BlockSpec((8, _D), lambda i: (0, 0)),
        ],
        out_specs=pl.BlockSpec((_BM, _D), lambda i: (i, 0)),
        out_shape=jax.ShapeDtypeStruct((_N, _D), jnp.float32),
    )


def kernel(x, edge_index, num_nodes, W, b, gamma, beta):
    x = x.astype(jnp.float32)
    ei = edge_index.astype(jnp.int32)
    src3 = ei[:, 0].reshape(_NW, _NCH, _CH)
    dst3 = ei[:, 1].reshape(_NW, _NCH, _CH)
    xa = jnp.concatenate(
        [x.astype(jnp.bfloat16),
         jnp.ones((_N, 1), jnp.bfloat16),
         jnp.zeros((_N, _DP - _D - 1), jnp.bfloat16)], axis=1)
    zsum = jnp.zeros((_RPT, _DP), jnp.bfloat16)
    psum = _get_sc_agg()(xa, src3, dst3, zsum)

    nz = (jnp.asarray(num_nodes, jnp.float32) - jnp.float32(_N)).reshape(1, 1)
    y, st = _get_stats_call()(
        nz, x, psum, W[:_D], W[_D:], b.reshape(1, _D),
        gamma.reshape(1, _D), beta.reshape(1, _D))
    return _get_norm_call()(y, st)


# speedup vs baseline: 1.1544x; 1.1544x over previous
import functools

import jax
import jax.numpy as jnp
from jax import lax
from jax.experimental import pallas as pl
from jax.experimental.pallas import tpu as pltpu
from jax.experimental.pallas import tpu_sc as plsc

_N = 10000
_E = 320000
_D = 128
_DC = 8
_NC = 2
_NS = 16
_NW = _NC * _NS
_EPW = _E // _NW
_CH = 40
_NCH = _EPW // _CH
_NP = 10112
_RPT = _NP // _NS
_BM = 2000
_NBLK = _N // _BM


def _sc_agg_body(x, src3, dst3, zsum, zcnt, ones_h, out_sum, out_cnt,
                 idx_s, idx_d, rows0, rows1, rows2, ones_v, acc, acc_c,
                 sem0, sem1, sem2, semc, semz):
    c = lax.axis_index("c")
    s = lax.axis_index("s")
    wid = s * _NC + c

    def gather(j, buf, sem):
        pltpu.async_copy(x.at[idx_s.at[j]], buf, sem)

    def wait_g(buf, sem):
        pltpu.make_async_copy(x.at[idx_s.at[0]], buf, sem).wait()

    def scat(j, buf):
        pltpu.sync_copy(buf, acc.at[idx_d.at[j]], add=True)
        pltpu.async_copy(ones_v, acc_c.at[idx_d.at[j]], semc, add=True)

    def drain_c(i, carry):
        pltpu.make_async_copy(ones_v, acc_c.at[idx_d.at[0]], semc).wait()
        return carry

    pltpu.sync_copy(src3.at[wid], idx_s)
    gather(0, rows0, sem0)
    gather(1, rows1, sem1)
    gather(2, rows2, sem2)
    pltpu.async_copy(dst3.at[wid], idx_d, semz)
    pltpu.async_copy(ones_h, ones_v, semz)
    pltpu.async_copy(zsum, acc.at[pl.ds(s * _RPT, _RPT)], semz)
    pltpu.async_copy(zcnt, acc_c.at[pl.ds(s * _RPT, _RPT)], semz)
    pltpu.make_async_copy(dst3.at[wid], idx_d, semz).wait()
    pltpu.make_async_copy(ones_h, ones_v, semz).wait()
    pltpu.make_async_copy(zsum, acc.at[pl.ds(s * _RPT, _RPT)], semz).wait()
    pltpu.make_async_copy(zcnt, acc_c.at[pl.ds(s * _RPT, _RPT)], semz).wait()
    plsc.subcore_barrier()

    def body(i, carry):
        j = 3 * i
        wait_g(rows0, sem0)
        scat(j, rows0)

        @pl.when(j + 3 < _NCH)
        def _pref0():
            gather(j + 3, rows0, sem0)

        wait_g(rows1, sem1)
        scat(j + 1, rows1)

        @pl.when(j + 4 < _NCH)
        def _pref1():
            gather(j + 4, rows1, sem1)

        wait_g(rows2, sem2)
        scat(j + 2, rows2)

        @pl.when(j + 5 < _NCH)
        def _pref2():
            gather(j + 5, rows2, sem2)

        return carry

    lax.fori_loop(0, _NCH // 3, body, 0)
    for r in range(_NCH - 3 * (_NCH // 3)):
        j = 3 * (_NCH // 3) + r
        buf, sem = [(rows0, sem0), (rows1, sem1), (rows2, sem2)][r]
        wait_g(buf, sem)
        scat(j, buf)

    lax.fori_loop(0, _NCH, drain_c, 0)
    plsc.subcore_barrier()
    pltpu.sync_copy(acc.at[pl.ds(s * _RPT, _RPT)],
                    out_sum.at[c, pl.ds(s * _RPT, _RPT)])
    pltpu.sync_copy(acc_c.at[pl.ds(s * _RPT, _RPT)],
                    out_cnt.at[c, pl.ds(s * _RPT, _RPT)])


@functools.lru_cache(maxsize=1)
def _get_sc_agg():
    return pl.kernel(
        _sc_agg_body,
        out_type=[
            jax.ShapeDtypeStruct((_NC, _NP, _D), jnp.float32),
            jax.ShapeDtypeStruct((_NC, _NP, _DC), jnp.float32),
        ],
        mesh=plsc.VectorSubcoreMesh(core_axis_name="c", subcore_axis_name="s"),
        scratch_types=[
            pltpu.VMEM((_NCH, _CH), jnp.int32),
            pltpu.VMEM((_NCH, _CH), jnp.int32),
            pltpu.VMEM((_CH, _D), jnp.float32),
            pltpu.VMEM((_CH, _D), jnp.float32),
            pltpu.VMEM((_CH, _D), jnp.float32),
            pltpu.VMEM((_CH, _DC), jnp.float32),
            pltpu.VMEM_SHARED((_NP, _D), jnp.float32),
            pltpu.VMEM_SHARED((_NP, _DC), jnp.float32),
            pltpu.SemaphoreType.DMA,
            pltpu.SemaphoreType.DMA,
            pltpu.SemaphoreType.DMA,
            pltpu.SemaphoreType.DMA,
            pltpu.SemaphoreType.DMA,
        ],
        compiler_params=pltpu.CompilerParams(use_tc_tiling_on_sc=False),
    )


def _tc_stats_body(nz_ref, x_ref, p_ref, pc_ref, w1_ref, w2_ref, b_ref, g_ref,
                   bt_ref, y_ref, st_ref, acc_ref):
    i = pl.program_id(0)

    @pl.when(i == 0)
    def _init():
        acc_ref[...] = jnp.zeros_like(acc_ref)
        st_ref[...] = jnp.zeros_like(st_ref)

    ssum = p_ref[0] + p_ref[1] + nz_ref[0, 0]
    cnt = jnp.maximum(pc_ref[0, :, 0:1] + pc_ref[1, :, 0:1], 1.0)
    mean = ssum / cnt
    y = (jnp.dot(x_ref[...], w1_ref[...], preferred_element_type=jnp.float32)
         + jnp.dot(mean, w2_ref[...], preferred_element_type=jnp.float32)
         + b_ref[...])
    y_ref[...] = y
    acc_ref[0:1, :] = acc_ref[0:1, :] + jnp.sum(y, axis=0, keepdims=True)
    acc_ref[1:2, :] = acc_ref[1:2, :] + jnp.sum(y * y, axis=0, keepdims=True)

    @pl.when(i == pl.num_programs(0) - 1)
    def _fin():
        mu = acc_ref[0:1, :] * (1.0 / _N)
        var = acc_ref[1:2, :] * (1.0 / _N) - mu * mu
        rstd = lax.rsqrt(var + 1e-5)
        scale = g_ref[...] * rstd
        st_ref[0:1, :] = scale
        st_ref[1:2, :] = bt_ref[...] - mu * scale


@functools.lru_cache(maxsize=1)
def _get_stats_call():
    return pl.pallas_call(
        _tc_stats_body,
        grid=(_NBLK,),
        in_specs=[
            pl.BlockSpec(memory_space=pltpu.SMEM),
            pl.BlockSpec((_BM, _D), lambda i: (i, 0)),
            pl.BlockSpec((_NC, _BM, _D), lambda i: (0, i, 0)),
            pl.BlockSpec((_NC, _BM, _DC), lambda i: (0, i, 0)),
            pl.BlockSpec((_D, _D), lambda i: (0, 0)),
            pl.BlockSpec((_D, _D), lambda i: (0, 0)),
            pl.BlockSpec((1, _D), lambda i: (0, 0)),
            pl.BlockSpec((1, _D), lambda i: (0, 0)),
            pl.BlockSpec((1, _D), lambda i: (0, 0)),
        ],
        out_specs=[
            pl.BlockSpec((_BM, _D), lambda i: (i, 0)),
            pl.BlockSpec((8, _D), lambda i: (0, 0)),
        ],
        out_shape=[
            jax.ShapeDtypeStruct((_N, _D), jnp.float32),
            jax.ShapeDtypeStruct((8, _D), jnp.float32),
        ],
        scratch_shapes=[pltpu.VMEM((8, _D), jnp.float32)],
        compiler_params=pltpu.CompilerParams(
            dimension_semantics=("arbitrary",)),
    )


def _tc_norm_body(y_ref, st_ref, o_ref):
    o_ref[...] = jnp.maximum(y_ref[...] * st_ref[0:1, :] + st_ref[1:2, :], 0.0)


@functools.lru_cache(maxsize=1)
def _get_norm_call():
    return pl.pallas_call(
        _tc_norm_body,
        grid=(_NBLK,),
        in_specs=[
            pl.BlockSpec((_BM, _D), lambda i: (i, 0)),
            pl.BlockSpec((8, _D), lambda i: (0, 0)),
        ],
        out_specs=pl.BlockSpec((_BM, _D), lambda i: (i, 0)),
        out_shape=jax.ShapeDtypeStruct((_N, _D), jnp.float32),
    )


def kernel(x, edge_index, num_nodes, W, b, gamma, beta):
    x = x.astype(jnp.float32)
    eit = edge_index.astype(jnp.int32).T
    src3 = eit[0].reshape(_NW, _NCH, _CH)
    dst3 = eit[1].reshape(_NW, _NCH, _CH)
    zsum = jnp.zeros((_RPT, _D), jnp.float32)
    zcnt = jnp.zeros((_RPT, _DC), jnp.float32)
    ones_h = jnp.ones((_CH, _DC), jnp.float32)
    psum, pcnt = _get_sc_agg()(x, src3, dst3, zsum, zcnt, ones_h)

    nz = (jnp.asarray(num_nodes, jnp.float32) - jnp.float32(_N)).reshape(1, 1)
    y, st = _get_stats_call()(
        nz, x, psum, pcnt, W[:_D], W[_D:], b.reshape(1, _D),
        gamma.reshape(1, _D), beta.reshape(1, _D))
    return _get_norm_call()(y, st)
